# baseline (speedup 1.0000x reference)
import functools

import jax
import jax.numpy as jnp
import numpy as np
from jax.experimental import pallas as pl
from jax.experimental.pallas import tpu as pltpu

B, N, S = 2, 8192, 2048
LOW, HIGH = 512, 256
NT = 1024
NCH = 1
KD = 32


def _trunc(x):
    u = jax.lax.bitcast_convert_type(x, jnp.uint32)
    return jax.lax.bitcast_convert_type(u & np.uint32(0xFFFF0000), jnp.float32)


def _split3(x):
    hi = _trunc(x)
    r1 = x - hi
    mid = _trunc(r1)
    r2 = r1 - mid
    lo = _trunc(r2)
    return (hi.astype(jnp.bfloat16), mid.astype(jnp.bfloat16),
            lo.astype(jnp.bfloat16))


def _tile_kernel(a_ref, b_ref, xh_ref, x_ref, ph_ref, w_ref, bias_ref,
                 out_ref, pl_scratch):
    n_idx = pl.program_id(1)

    @pl.when(n_idx == 0)
    def _per_batch():
        acc = jnp.dot(w_ref[...], x_ref[0], preferred_element_type=jnp.float32)
        pl_scratch[...] = jnp.maximum(acc + bias_ref[...], 0.0)

    e = jnp.dot(a_ref[0], b_ref[0], preferred_element_type=jnp.float32)

    inf = jnp.float32(np.inf)
    e4 = e.reshape(S // (8 * NCH), NCH, 8, NT)
    big = jnp.full((8, NT), inf, jnp.float32)
    chains = [(big, big, big) for _ in range(NCH)]

    def _insert(t, v):
        a, b, c = t
        na = jnp.minimum(a, v)
        t1 = jnp.maximum(a, v)
        nb = jnp.minimum(b, t1)
        t2 = jnp.maximum(b, t1)
        nc = jnp.minimum(c, t2)
        return na, nb, nc

    for i in range(S // (8 * NCH)):
        chains = [_insert(chains[k], e4[i, k]) for k in range(NCH)]

    def _merge(t1, t2):
        t = _insert(t1, t2[0])
        t = _insert(t, t2[1])
        return _insert(t, t2[2])

    t = chains[0]
    for k in range(1, NCH):
        t = _merge(t, chains[k])
    a, b, c = t

    cat = jnp.concatenate([a, b, c], axis=0)
    m0 = jnp.min(cat, axis=0, keepdims=True)
    c1 = jnp.where(cat > m0, cat, inf)
    m1 = jnp.min(c1, axis=0, keepdims=True)
    c2 = jnp.where(c1 > m1, c1, inf)
    m2 = jnp.min(c2, axis=0, keepdims=True)

    xh = xh_ref[0]
    nh = (xh[0:1, :] * xh[0:1, :] + xh[1:2, :] * xh[1:2, :]
          + xh[2:3, :] * xh[2:3, :])
    r0 = 1.0 / (jnp.maximum(m0 + nh, 0.0) + 1e-8)
    r1 = 1.0 / (jnp.maximum(m1 + nh, 0.0) + 1e-8)
    r2 = 1.0 / (jnp.maximum(m2 + nh, 0.0) + 1e-8)
    norm = r0 + r1 + r2
    w0 = r0 / norm
    w1 = r1 / norm
    w2 = r2 / norm

    zero = jnp.float32(0.0)
    wsp = jnp.where(e == m0, w0,
                    jnp.where(e == m1, w1,
                              jnp.where(e == m2, w2, zero)))

    interp = jnp.dot(pl_scratch[...], wsp, preferred_element_type=jnp.float32)
    out_ref[0] = interp + ph_ref[0]


@jax.jit
def kernel(xyz_low, xyz_high, points_low, points_high, W, b, gamma, beta,
           running_mean, running_var):
    scale = gamma / jnp.sqrt(running_var + 1e-5)
    w_folded = W * scale[:, None]
    b_folded = ((b - running_mean) * scale + beta)[:, None]

    xl_t = jnp.transpose(xyz_low, (0, 2, 1))
    a_limbs = _split3(-2.0 * xl_t)
    b_limbs = _split3(xyz_high)
    nl = jnp.sum(xl_t * xl_t, axis=2, keepdims=True)
    nl_limbs = _split3(nl)

    a_cols = []
    b_rows = []
    for ai in a_limbs:
        for bj in b_limbs:
            a_cols.append(ai)
            b_rows.append(bj)
    a_pack = jnp.concatenate(a_cols + list(nl_limbs), axis=2)
    ones_row = jnp.ones((B, 1, N), jnp.bfloat16)
    b_pack = jnp.concatenate(b_rows + [ones_row] * 3, axis=1)
    pad_a = jnp.zeros((B, S, KD - a_pack.shape[2]), jnp.bfloat16)
    pad_b = jnp.zeros((B, KD - b_pack.shape[1], N), jnp.bfloat16)
    a_pack = jnp.concatenate([a_pack, pad_a], axis=2)
    b_pack = jnp.concatenate([b_pack, pad_b], axis=1)

    grid = (B, N // NT)
    out = pl.pallas_call(
        _tile_kernel,
        grid=grid,
        in_specs=[
            pl.BlockSpec((1, S, KD), lambda bi, ni: (bi, 0, 0)),
            pl.BlockSpec((1, KD, NT), lambda bi, ni: (bi, 0, ni)),
            pl.BlockSpec((1, 3, NT), lambda bi, ni: (bi, 0, ni)),
            pl.BlockSpec((1, LOW, S), lambda bi, ni: (bi, 0, 0)),
            pl.BlockSpec((1, HIGH, NT), lambda bi, ni: (bi, 0, ni)),
            pl.BlockSpec((HIGH, LOW), lambda bi, ni: (0, 0)),
            pl.BlockSpec((HIGH, 1), lambda bi, ni: (0, 0)),
        ],
        out_specs=pl.BlockSpec((1, HIGH, NT), lambda bi, ni: (bi, 0, ni)),
        out_shape=jax.ShapeDtypeStruct((B, HIGH, N), jnp.float32),
        scratch_shapes=[
            pltpu.VMEM((HIGH, S), jnp.float32),
        ],
    )(a_pack, b_pack, xyz_high, points_low, points_high, w_folded, b_folded)
    return out


# speedup vs baseline: 1.0199x; 1.0199x over previous
import functools

import jax
import jax.numpy as jnp
import numpy as np
from jax.experimental import pallas as pl
from jax.experimental.pallas import tpu as pltpu

B, N, S = 2, 8192, 2048
LOW, HIGH = 512, 256
NT = 2048
NCH = 1
KD = 32


def _trunc(x):
    u = jax.lax.bitcast_convert_type(x, jnp.uint32)
    return jax.lax.bitcast_convert_type(u & np.uint32(0xFFFF0000), jnp.float32)


def _split3(x):
    hi = _trunc(x)
    r1 = x - hi
    mid = _trunc(r1)
    r2 = r1 - mid
    lo = _trunc(r2)
    return (hi.astype(jnp.bfloat16), mid.astype(jnp.bfloat16),
            lo.astype(jnp.bfloat16))


def _tile_kernel(a_ref, b_ref, xh_ref, x_ref, ph_ref, w_ref, bias_ref,
                 out_ref, pl_scratch):
    n_idx = pl.program_id(1)

    @pl.when(n_idx == 0)
    def _per_batch():
        acc = jnp.dot(w_ref[...], x_ref[0], preferred_element_type=jnp.float32)
        pl_scratch[...] = jnp.maximum(acc + bias_ref[...], 0.0)

    e = jnp.dot(a_ref[0], b_ref[0], preferred_element_type=jnp.float32)

    inf = jnp.float32(np.inf)
    e4 = e.reshape(S // (8 * NCH), NCH, 8, NT)
    big = jnp.full((8, NT), inf, jnp.float32)
    chains = [(big, big, big) for _ in range(NCH)]

    def _insert(t, v):
        a, b, c = t
        na = jnp.minimum(a, v)
        t1 = jnp.maximum(a, v)
        nb = jnp.minimum(b, t1)
        t2 = jnp.maximum(b, t1)
        nc = jnp.minimum(c, t2)
        return na, nb, nc

    for i in range(S // (8 * NCH)):
        chains = [_insert(chains[k], e4[i, k]) for k in range(NCH)]

    def _merge(t1, t2):
        t = _insert(t1, t2[0])
        t = _insert(t, t2[1])
        return _insert(t, t2[2])

    t = chains[0]
    for k in range(1, NCH):
        t = _merge(t, chains[k])
    a, b, c = t

    cat = jnp.concatenate([a, b, c], axis=0)
    m0 = jnp.min(cat, axis=0, keepdims=True)
    c1 = jnp.where(cat > m0, cat, inf)
    m1 = jnp.min(c1, axis=0, keepdims=True)
    c2 = jnp.where(c1 > m1, c1, inf)
    m2 = jnp.min(c2, axis=0, keepdims=True)

    xh = xh_ref[0]
    nh = (xh[0:1, :] * xh[0:1, :] + xh[1:2, :] * xh[1:2, :]
          + xh[2:3, :] * xh[2:3, :])
    r0 = 1.0 / (jnp.maximum(m0 + nh, 0.0) + 1e-8)
    r1 = 1.0 / (jnp.maximum(m1 + nh, 0.0) + 1e-8)
    r2 = 1.0 / (jnp.maximum(m2 + nh, 0.0) + 1e-8)
    norm = r0 + r1 + r2
    w0 = r0 / norm
    w1 = r1 / norm
    w2 = r2 / norm

    zero = jnp.float32(0.0)
    wsp = jnp.where(e == m0, w0,
                    jnp.where(e == m1, w1,
                              jnp.where(e == m2, w2, zero)))

    interp = jnp.dot(pl_scratch[...], wsp, preferred_element_type=jnp.float32)
    out_ref[0] = interp + ph_ref[0]


@jax.jit
def kernel(xyz_low, xyz_high, points_low, points_high, W, b, gamma, beta,
           running_mean, running_var):
    scale = gamma / jnp.sqrt(running_var + 1e-5)
    w_folded = W * scale[:, None]
    b_folded = ((b - running_mean) * scale + beta)[:, None]

    xl_t = jnp.transpose(xyz_low, (0, 2, 1))
    a_limbs = _split3(-2.0 * xl_t)
    b_limbs = _split3(xyz_high)
    nl = jnp.sum(xl_t * xl_t, axis=2, keepdims=True)
    nl_limbs = _split3(nl)

    a_cols = []
    b_rows = []
    for ai in a_limbs:
        for bj in b_limbs:
            a_cols.append(ai)
            b_rows.append(bj)
    a_pack = jnp.concatenate(a_cols + list(nl_limbs), axis=2)
    ones_row = jnp.ones((B, 1, N), jnp.bfloat16)
    b_pack = jnp.concatenate(b_rows + [ones_row] * 3, axis=1)
    pad_a = jnp.zeros((B, S, KD - a_pack.shape[2]), jnp.bfloat16)
    pad_b = jnp.zeros((B, KD - b_pack.shape[1], N), jnp.bfloat16)
    a_pack = jnp.concatenate([a_pack, pad_a], axis=2)
    b_pack = jnp.concatenate([b_pack, pad_b], axis=1)

    grid = (B, N // NT)
    out = pl.pallas_call(
        _tile_kernel,
        grid=grid,
        in_specs=[
            pl.BlockSpec((1, S, KD), lambda bi, ni: (bi, 0, 0)),
            pl.BlockSpec((1, KD, NT), lambda bi, ni: (bi, 0, ni)),
            pl.BlockSpec((1, 3, NT), lambda bi, ni: (bi, 0, ni)),
            pl.BlockSpec((1, LOW, S), lambda bi, ni: (bi, 0, 0)),
            pl.BlockSpec((1, HIGH, NT), lambda bi, ni: (bi, 0, ni)),
            pl.BlockSpec((HIGH, LOW), lambda bi, ni: (0, 0)),
            pl.BlockSpec((HIGH, 1), lambda bi, ni: (0, 0)),
        ],
        out_specs=pl.BlockSpec((1, HIGH, NT), lambda bi, ni: (bi, 0, ni)),
        out_shape=jax.ShapeDtypeStruct((B, HIGH, N), jnp.float32),
        scratch_shapes=[
            pltpu.VMEM((HIGH, S), jnp.float32),
        ],
    )(a_pack, b_pack, xyz_high, points_low, points_high, w_folded, b_folded)
    return out
